# Initial kernel scaffold; baseline (speedup 1.0000x reference)
import jax
import jax.numpy as jnp
from jax.experimental import pallas as pl


def kernel(x_s, edge_index_s, x_c, edge_index_c, Q, w1s, b1s, w2s, b2s, w1c, b1c, w2c, b2c, attn_w1, attn_b1, attn_w2, attn_b2, cls_w, cls_b, dd_w1, dd_b1, dd_w2, dd_b2):
    raise NotImplementedError("write your pallas kernel here")


# speedup vs baseline: 7.1431x; 7.1431x over previous
import functools

import jax
import jax.numpy as jnp
from jax import lax
from jax.experimental import pallas as pl
from jax.experimental.pallas import tpu as pltpu
from jax.experimental.pallas import tpu_sc as plsc

N_S, N_C = 50000, 500
E_S, E_C = 800000, 8000
NP_S, NP_C = 51200, 2048
EP_S, EP_C = 802816, 8192
RS, RC = EP_S // 128, EP_C // 128
TS_S, TS_C = NP_S // 16, NP_C // 16
RW_S, RW_C = RS // 32, RC // 32
RB_S, RB_C = RS // 16, RC // 16

_MESH = plsc.VectorSubcoreMesh(core_axis_name="c", subcore_axis_name="s")


def _f32(*shape):
    return jax.ShapeDtypeStruct(shape, jnp.float32)


@functools.partial(
    pl.kernel,
    out_type=[_f32(2 * NP_S), _f32(2 * NP_C)],
    mesh=_MESH,
    scratch_types=[
        pltpu.VMEM_SHARED((NP_S,), jnp.float32),
        pltpu.VMEM_SHARED((NP_C,), jnp.float32),
        pltpu.VMEM((128,), jnp.int32),
        pltpu.VMEM((128,), jnp.float32),
    ],
)
def _sc_deg(dst_s, dst_c, zrow, deg_s, deg_c, acc_s, acc_c, didx, ones_v):
    c = lax.axis_index("c")
    s = lax.axis_index("s")
    w = c * 16 + s
    for j in range(8):
        ones_v[pl.ds(j * 16, 16)] = jnp.ones((16,), jnp.float32)
    pltpu.sync_copy(zrow, acc_s.at[pl.ds(s * TS_S, TS_S)])
    pltpu.sync_copy(zrow.at[pl.ds(0, TS_C)], acc_c.at[pl.ds(s * TS_C, TS_C)])
    plsc.subcore_barrier()

    def body(r, carry):
        pltpu.sync_copy(dst_s.at[pl.ds((w * RW_S + r) * 128, 128)], didx)
        pltpu.sync_copy(ones_v, acc_s.at[didx], add=True)
        return carry

    lax.fori_loop(0, RW_S, body, 0)
    for r in range(RW_C):
        pltpu.sync_copy(dst_c.at[pl.ds((w * RW_C + r) * 128, 128)], didx)
        pltpu.sync_copy(ones_v, acc_c.at[didx], add=True)
    plsc.subcore_barrier()
    pltpu.sync_copy(acc_s.at[pl.ds(s * TS_S, TS_S)],
                    deg_s.at[pl.ds(c * NP_S + s * TS_S, TS_S)])
    pltpu.sync_copy(acc_c.at[pl.ds(s * TS_C, TS_C)],
                    deg_c.at[pl.ds(c * NP_C + s * TS_C, TS_C)])


@functools.partial(
    pl.kernel,
    out_type=[_f32(8 * NP_S), _f32(8 * NP_C)],
    mesh=_MESH,
    scratch_types=[
        pltpu.VMEM_SHARED((4 * NP_S,), jnp.float32),
        pltpu.VMEM_SHARED((4 * NP_S,), jnp.float32),
        pltpu.VMEM_SHARED((4 * NP_C,), jnp.float32),
        pltpu.VMEM_SHARED((4 * NP_C,), jnp.float32),
        pltpu.VMEM((128,), jnp.int32),
        pltpu.VMEM((128,), jnp.int32),
        pltpu.VMEM((4, 128), jnp.float32),
        pltpu.SemaphoreType.DMA,
        pltpu.SemaphoreType.DMA,
    ],
)
def _sc_seg1(y1, y1c, src_s, dst_s, src_c, dst_c, zrow, agg, aggc,
             stbl, acc, stblc, accc, sidx, didx, rows, gsem, ssem):
    c = lax.axis_index("c")
    s = lax.axis_index("s")
    p0 = c * 4
    for f in range(4):
        pltpu.sync_copy(y1.at[pl.ds((p0 + f) * NP_S + s * TS_S, TS_S)],
                        stbl.at[pl.ds(f * NP_S + s * TS_S, TS_S)])
        pltpu.sync_copy(zrow, acc.at[pl.ds(f * NP_S + s * TS_S, TS_S)])
        pltpu.sync_copy(y1c.at[pl.ds((p0 + f) * NP_C + s * TS_C, TS_C)],
                        stblc.at[pl.ds(f * NP_C + s * TS_C, TS_C)])
        pltpu.sync_copy(zrow.at[pl.ds(0, TS_C)],
                        accc.at[pl.ds(f * NP_C + s * TS_C, TS_C)])
    plsc.subcore_barrier()

    def body(r, carry):
        base = (s * RB_S + r) * 128
        pltpu.sync_copy(src_s.at[pl.ds(base, 128)], sidx)
        pltpu.sync_copy(dst_s.at[pl.ds(base, 128)], didx)
        gets = [pltpu.async_copy(stbl.at[pl.ds(f * NP_S, NP_S)].at[sidx],
                                 rows.at[f], gsem) for f in range(4)]
        for g in gets:
            g.wait()
        puts = [pltpu.async_copy(rows.at[f],
                                 acc.at[pl.ds(f * NP_S, NP_S)].at[didx],
                                 ssem, add=True) for f in range(4)]
        for p in puts:
            p.wait()
        return carry

    lax.fori_loop(0, RB_S, body, 0)
    for r in range(RB_C):
        base = (s * RB_C + r) * 128
        pltpu.sync_copy(src_c.at[pl.ds(base, 128)], sidx)
        pltpu.sync_copy(dst_c.at[pl.ds(base, 128)], didx)
        gets = [pltpu.async_copy(stblc.at[pl.ds(f * NP_C, NP_C)].at[sidx],
                                 rows.at[f], gsem) for f in range(4)]
        for g in gets:
            g.wait()
        puts = [pltpu.async_copy(rows.at[f],
                                 accc.at[pl.ds(f * NP_C, NP_C)].at[didx],
                                 ssem, add=True) for f in range(4)]
        for p in puts:
            p.wait()
    plsc.subcore_barrier()
    for f in range(4):
        pltpu.sync_copy(acc.at[pl.ds(f * NP_S + s * TS_S, TS_S)],
                        agg.at[pl.ds((p0 + f) * NP_S + s * TS_S, TS_S)])
        pltpu.sync_copy(accc.at[pl.ds(f * NP_C + s * TS_C, TS_C)],
                        aggc.at[pl.ds((p0 + f) * NP_C + s * TS_C, TS_C)])


@functools.partial(
    pl.kernel,
    out_type=[_f32(64 * NP_S), _f32(64 * NP_C)],
    mesh=_MESH,
    scratch_types=[
        pltpu.VMEM_SHARED((16 * NP_S,), jnp.float32),
        pltpu.VMEM_SHARED((16 * NP_S,), jnp.float32),
        pltpu.VMEM_SHARED((16 * NP_C,), jnp.float32),
        pltpu.VMEM_SHARED((16 * NP_C,), jnp.float32),
        pltpu.VMEM((128,), jnp.int32),
        pltpu.VMEM((128,), jnp.int32),
        pltpu.VMEM((8, 128), jnp.float32),
        pltpu.SemaphoreType.DMA,
        pltpu.SemaphoreType.DMA,
    ],
)
def _sc_seg2(y2, y2c, src_s, dst_s, src_c, dst_c, zrow, agg, aggc,
             stbl, acc, stblc, accc, sidx, didx, rows, gsem, ssem):
    c = lax.axis_index("c")
    s = lax.axis_index("s")
    for g in range(2):
        p0 = c * 32 + g * 16
        for f in range(16):
            pltpu.sync_copy(y2.at[pl.ds((p0 + f) * NP_S + s * TS_S, TS_S)],
                            stbl.at[pl.ds(f * NP_S + s * TS_S, TS_S)])
            pltpu.sync_copy(zrow, acc.at[pl.ds(f * NP_S + s * TS_S, TS_S)])
            pltpu.sync_copy(y2c.at[pl.ds((p0 + f) * NP_C + s * TS_C, TS_C)],
                            stblc.at[pl.ds(f * NP_C + s * TS_C, TS_C)])
            pltpu.sync_copy(zrow.at[pl.ds(0, TS_C)],
                            accc.at[pl.ds(f * NP_C + s * TS_C, TS_C)])
        plsc.subcore_barrier()

        def body(r, carry):
            base = (s * RB_S + r) * 128
            pltpu.sync_copy(src_s.at[pl.ds(base, 128)], sidx)
            pltpu.sync_copy(dst_s.at[pl.ds(base, 128)], didx)

            def half(h, carry2):
                gets = [pltpu.async_copy(
                    stbl.at[pl.ds((h * 8 + f) * NP_S, NP_S)].at[sidx],
                    rows.at[f], gsem) for f in range(8)]
                for gg in gets:
                    gg.wait()
                puts = [pltpu.async_copy(
                    rows.at[f],
                    acc.at[pl.ds((h * 8 + f) * NP_S, NP_S)].at[didx],
                    ssem, add=True) for f in range(8)]
                for p in puts:
                    p.wait()
                return carry2

            lax.fori_loop(0, 2, half, 0)
            return carry

        lax.fori_loop(0, RB_S, body, 0)
        for r in range(RB_C):
            base = (s * RB_C + r) * 128
            pltpu.sync_copy(src_c.at[pl.ds(base, 128)], sidx)
            pltpu.sync_copy(dst_c.at[pl.ds(base, 128)], didx)

            def chalf(h, carry2):
                gets = [pltpu.async_copy(
                    stblc.at[pl.ds((h * 8 + f) * NP_C, NP_C)].at[sidx],
                    rows.at[f], gsem) for f in range(8)]
                for gg in gets:
                    gg.wait()
                puts = [pltpu.async_copy(
                    rows.at[f],
                    accc.at[pl.ds((h * 8 + f) * NP_C, NP_C)].at[didx],
                    ssem, add=True) for f in range(8)]
                for p in puts:
                    p.wait()
                return carry2

            lax.fori_loop(0, 2, chalf, 0)
        plsc.subcore_barrier()
        for f in range(16):
            pltpu.sync_copy(acc.at[pl.ds(f * NP_S + s * TS_S, TS_S)],
                            agg.at[pl.ds((p0 + f) * NP_S + s * TS_S, TS_S)])
            pltpu.sync_copy(accc.at[pl.ds(f * NP_C + s * TS_C, TS_C)],
                            aggc.at[pl.ds((p0 + f) * NP_C + s * TS_C, TS_C)])


def _tc_prep_body(deg_ref, xT_ref, dinv_ref, y1_ref):
    deg = deg_ref[0:1] + deg_ref[1:2] + 1.0
    dinv = lax.rsqrt(deg)
    dinv_ref[...] = dinv
    y1_ref[...] = dinv * xT_ref[...]


def _tc_prep(degp, xT, n_pad, blk):
    return pl.pallas_call(
        _tc_prep_body,
        grid=(n_pad // blk,),
        in_specs=[
            pl.BlockSpec((2, blk), lambda i: (0, i)),
            pl.BlockSpec((8, blk), lambda i: (0, i)),
        ],
        out_specs=[
            pl.BlockSpec((1, blk), lambda i: (0, i)),
            pl.BlockSpec((8, blk), lambda i: (0, i)),
        ],
        out_shape=[_f32(1, n_pad), _f32(8, n_pad)],
    )(degp, xT)


def _tc_l1_body(dinv_ref, agg_ref, y1_ref, w_ref, b_ref, y2_ref):
    pre = dinv_ref[...] * (agg_ref[...] + y1_ref[...])
    h = lax.dot_general(w_ref[...], pre, (((0,), (0,)), ((), ())),
                        preferred_element_type=jnp.float32)
    xs1 = jnp.maximum(h + b_ref[...], 0.0)
    y2_ref[...] = dinv_ref[...] * xs1


def _tc_l1(dinv, agg, y1, w_pad, b, n_pad, blk):
    return pl.pallas_call(
        _tc_l1_body,
        grid=(n_pad // blk,),
        in_specs=[
            pl.BlockSpec((1, blk), lambda i: (0, i)),
            pl.BlockSpec((8, blk), lambda i: (0, i)),
            pl.BlockSpec((8, blk), lambda i: (0, i)),
            pl.BlockSpec((8, 64), lambda i: (0, 0)),
            pl.BlockSpec((64, 1), lambda i: (0, 0)),
        ],
        out_specs=pl.BlockSpec((64, blk), lambda i: (0, i)),
        out_shape=_f32(64, n_pad),
    )(dinv, agg, y1, w_pad, b)


def _tc_l2c_body(dinv_ref, agg_ref, y2_ref, w_ref, b_ref, xc2_ref):
    pre = dinv_ref[...] * (agg_ref[...] + y2_ref[...])
    xc2 = jnp.maximum(
        lax.dot_general(pre, w_ref[...], (((0,), (0,)), ((), ())),
                        preferred_element_type=jnp.float32) + b_ref[...], 0.0)
    xc2_ref[...] = xc2[:512]


def _tc_l2c(dinv_c, agg2c, y2c, w2c, b2c):
    return pl.pallas_call(
        _tc_l2c_body,
        out_shape=_f32(512, 128),
    )(dinv_c, agg2c, y2c, w2c, b2c)


def _tc_final_body(dinv_ref, agg_ref, y2_ref, w2_ref, b2_ref, q_ref, xc2_ref,
                   aw1_ref, ab1_ref, aw2_ref, ab2_ref, cw_ref, cb_ref,
                   dw1_ref, db1_ref, dw2_ref, db2_ref,
                   cls_ref, dom_ref, fused_ref):
    pre = dinv_ref[...] * (agg_ref[...] + y2_ref[...])
    xs2 = jnp.maximum(
        lax.dot_general(pre, w2_ref[...], (((0,), (0,)), ((), ())),
                        preferred_element_type=jnp.float32) + b2_ref[...], 0.0)
    q = q_ref[...]
    qp = jnp.concatenate(
        [q, jnp.zeros((q.shape[0], 512 - N_C), jnp.float32)], axis=1)
    xc_dec = jnp.dot(qp, xc2_ref[...], preferred_element_type=jnp.float32)
    aw1 = aw1_ref[...]
    a = jnp.maximum(
        jnp.dot(xs2, aw1[:128], preferred_element_type=jnp.float32)
        + jnp.dot(xc_dec, aw1[128:], preferred_element_type=jnp.float32)
        + ab1_ref[...], 0.0)
    logits = jnp.dot(a, aw2_ref[...], preferred_element_type=jnp.float32) + ab2_ref[...]
    m = jnp.max(logits, axis=1, keepdims=True)
    e = jnp.exp(logits - m)
    wgt = e / jnp.sum(e, axis=1, keepdims=True)
    fused = wgt[:, 0:1] * xs2 + wgt[:, 1:2] * xc_dec
    cls_ref[...] = jnp.dot(fused, cw_ref[...],
                           preferred_element_type=jnp.float32) + cb_ref[...]
    d = jnp.maximum(jnp.dot(fused, dw1_ref[...],
                            preferred_element_type=jnp.float32) + db1_ref[...], 0.0)
    dom_ref[...] = jnp.dot(d, dw2_ref[...],
                           preferred_element_type=jnp.float32) + db2_ref[...]
    fused_ref[...] = fused


def _tc_final(dinv, agg2, y2, w2s, b2s, Q, xc2,
              attn_w1, attn_b1, attn_w2, attn_b2, cls_w, cls_b,
              dd_w1, dd_b1, dd_w2, dd_b2, blk):
    def full(shape):
        return pl.BlockSpec(shape, lambda i: tuple(0 for _ in shape))

    return pl.pallas_call(
        _tc_final_body,
        grid=((N_S + blk - 1) // blk,),
        in_specs=[
            pl.BlockSpec((1, blk), lambda i: (0, i)),
            pl.BlockSpec((64, blk), lambda i: (0, i)),
            pl.BlockSpec((64, blk), lambda i: (0, i)),
            full((64, 128)), full((1, 128)),
            pl.BlockSpec((blk, N_C), lambda i: (i, 0)),
            full((512, 128)),
            full((256, 128)), full((1, 128)),
            full((128, 2)), full((1, 2)),
            full((128, 9)), full((1, 9)),
            full((128, 64)), full((1, 64)),
            full((64, 2)), full((1, 2)),
        ],
        out_specs=[
            pl.BlockSpec((blk, 9), lambda i: (i, 0)),
            pl.BlockSpec((blk, 2), lambda i: (i, 0)),
            pl.BlockSpec((blk, 128), lambda i: (i, 0)),
        ],
        out_shape=[_f32(N_S, 9), _f32(N_S, 2), _f32(N_S, 128)],
    )(dinv, agg2, y2, w2s, b2s, Q, xc2,
      attn_w1, attn_b1, attn_w2, attn_b2, cls_w, cls_b,
      dd_w1, dd_b1, dd_w2, dd_b2)


def _pad_edges(edge_index, e_pad, n_real, n_pad):
    e_real = edge_index.shape[1]
    npad = e_pad - e_real
    njunk = n_pad - n_real
    fill_s = n_real + (jnp.arange(npad, dtype=jnp.int32) % njunk)
    fill_d = n_real + ((jnp.arange(npad, dtype=jnp.int32) * 7) % njunk)
    src = jnp.concatenate([edge_index[0], fill_s])
    dst = jnp.concatenate([edge_index[1], fill_d])
    return src, dst


def kernel(x_s, edge_index_s, x_c, edge_index_c, Q,
           w1s, b1s, w2s, b2s, w1c, b1c, w2c, b2c,
           attn_w1, attn_b1, attn_w2, attn_b2,
           cls_w, cls_b, dd_w1, dd_b1, dd_w2, dd_b2):
    f32 = jnp.float32
    src_s, dst_s = _pad_edges(edge_index_s, EP_S, N_S, NP_S)
    src_c, dst_c = _pad_edges(edge_index_c, EP_C, N_C, NP_C)
    xT_s = jnp.pad(x_s, ((0, NP_S - N_S), (0, 1))).T
    xT_c = jnp.pad(x_c, ((0, NP_C - N_C), (0, 1))).T
    w1s_pad = jnp.pad(w1s, ((0, 1), (0, 0)))
    w1c_pad = jnp.pad(w1c, ((0, 1), (0, 0)))
    zrow = jnp.zeros((TS_S,), f32)

    degp_s, degp_c = _sc_deg(dst_s, dst_c, zrow)

    dinv_s, y1s = _tc_prep(degp_s.reshape(2, NP_S), xT_s, NP_S, 6400)
    dinv_c, y1c = _tc_prep(degp_c.reshape(2, NP_C), xT_c, NP_C, NP_C)

    agg1, agg1c = _sc_seg1(y1s.reshape(-1), y1c.reshape(-1),
                           src_s, dst_s, src_c, dst_c, zrow)

    y2 = _tc_l1(dinv_s, agg1.reshape(8, NP_S), y1s, w1s_pad,
                b1s.reshape(64, 1), NP_S, 3200)
    y2c = _tc_l1(dinv_c, agg1c.reshape(8, NP_C), y1c, w1c_pad,
                 b1c.reshape(64, 1), NP_C, NP_C)

    agg2, agg2c = _sc_seg2(y2.reshape(-1), y2c.reshape(-1),
                           src_s, dst_s, src_c, dst_c, zrow)

    xc2 = _tc_l2c(dinv_c, agg2c.reshape(64, NP_C), y2c, w2c,
                  b2c.reshape(1, -1))

    return _tc_final(dinv_s, agg2.reshape(64, NP_S), y2, w2s,
                     b2s.reshape(1, -1), Q, xc2,
                     attn_w1, attn_b1.reshape(1, -1),
                     attn_w2, attn_b2.reshape(1, -1),
                     cls_w, cls_b.reshape(1, -1),
                     dd_w1, dd_b1.reshape(1, -1),
                     dd_w2, dd_b2.reshape(1, -1), 1024)


# speedup vs baseline: 7.9604x; 1.1144x over previous
import functools

import jax
import jax.numpy as jnp
from jax import lax
from jax.experimental import pallas as pl
from jax.experimental.pallas import tpu as pltpu
from jax.experimental.pallas import tpu_sc as plsc

N_S, N_C = 50000, 500
E_S, E_C = 800000, 8000
NP_S, NP_C = 51200, 2048
EP_S, EP_C = 802816, 8192
RS, RC = EP_S // 128, EP_C // 128
TS_S, TS_C = NP_S // 16, NP_C // 16
RW_S, RW_C = RS // 32, RC // 32
RB_S, RB_C = RS // 16, RC // 16

_MESH = plsc.VectorSubcoreMesh(core_axis_name="c", subcore_axis_name="s")


def _f32(*shape):
    return jax.ShapeDtypeStruct(shape, jnp.float32)


@functools.partial(
    pl.kernel,
    out_type=[_f32(2 * NP_S), _f32(2 * NP_C)],
    mesh=_MESH,
    scratch_types=[
        pltpu.VMEM_SHARED((NP_S,), jnp.float32),
        pltpu.VMEM_SHARED((NP_C,), jnp.float32),
        pltpu.VMEM((128,), jnp.int32),
        pltpu.VMEM((128,), jnp.float32),
    ],
)
def _sc_deg(dst_s, dst_c, zrow, deg_s, deg_c, acc_s, acc_c, didx, ones_v):
    c = lax.axis_index("c")
    s = lax.axis_index("s")
    w = c * 16 + s
    for j in range(8):
        ones_v[pl.ds(j * 16, 16)] = jnp.ones((16,), jnp.float32)
    pltpu.sync_copy(zrow, acc_s.at[pl.ds(s * TS_S, TS_S)])
    pltpu.sync_copy(zrow.at[pl.ds(0, TS_C)], acc_c.at[pl.ds(s * TS_C, TS_C)])
    plsc.subcore_barrier()

    def body(r, carry):
        pltpu.sync_copy(dst_s.at[pl.ds((w * RW_S + r) * 128, 128)], didx)
        pltpu.sync_copy(ones_v, acc_s.at[didx], add=True)
        return carry

    lax.fori_loop(0, RW_S, body, 0)
    for r in range(RW_C):
        pltpu.sync_copy(dst_c.at[pl.ds((w * RW_C + r) * 128, 128)], didx)
        pltpu.sync_copy(ones_v, acc_c.at[didx], add=True)
    plsc.subcore_barrier()
    pltpu.sync_copy(acc_s.at[pl.ds(s * TS_S, TS_S)],
                    deg_s.at[pl.ds(c * NP_S + s * TS_S, TS_S)])
    pltpu.sync_copy(acc_c.at[pl.ds(s * TS_C, TS_C)],
                    deg_c.at[pl.ds(c * NP_C + s * TS_C, TS_C)])


@functools.partial(
    pl.kernel,
    out_type=[_f32(8 * NP_S), _f32(8 * NP_C)],
    mesh=_MESH,
    scratch_types=[
        pltpu.VMEM_SHARED((4 * NP_S,), jnp.float32),
        pltpu.VMEM_SHARED((4 * NP_S,), jnp.float32),
        pltpu.VMEM_SHARED((4 * NP_C,), jnp.float32),
        pltpu.VMEM_SHARED((4 * NP_C,), jnp.float32),
        pltpu.VMEM((128,), jnp.int32),
        pltpu.VMEM((128,), jnp.int32),
        pltpu.VMEM((4, 128), jnp.float32),
        pltpu.SemaphoreType.DMA,
        pltpu.SemaphoreType.DMA,
        pltpu.SemaphoreType.DMA,
        pltpu.SemaphoreType.DMA,
        pltpu.SemaphoreType.DMA,
        pltpu.SemaphoreType.DMA,
        pltpu.SemaphoreType.DMA,
        pltpu.SemaphoreType.DMA,
    ],
)
def _sc_seg1(y1, y1c, src_s, dst_s, src_c, dst_c, zrow, agg, aggc,
             stbl, acc, stblc, accc, sidx, didx, rows,
             gs0, gs1, gs2, gs3, ss0, ss1, ss2, ss3):
    gsem = [gs0, gs1, gs2, gs3]
    ssem = [ss0, ss1, ss2, ss3]
    c = lax.axis_index("c")
    s = lax.axis_index("s")
    p0 = c * 4
    for f in range(4):
        pltpu.sync_copy(y1.at[pl.ds((p0 + f) * NP_S + s * TS_S, TS_S)],
                        stbl.at[pl.ds(f * NP_S + s * TS_S, TS_S)])
        pltpu.sync_copy(zrow, acc.at[pl.ds(f * NP_S + s * TS_S, TS_S)])
        pltpu.sync_copy(y1c.at[pl.ds((p0 + f) * NP_C + s * TS_C, TS_C)],
                        stblc.at[pl.ds(f * NP_C + s * TS_C, TS_C)])
        pltpu.sync_copy(zrow.at[pl.ds(0, TS_C)],
                        accc.at[pl.ds(f * NP_C + s * TS_C, TS_C)])
    plsc.subcore_barrier()

    def body(r, carry):
        base = (s * RB_S + r) * 128
        pltpu.sync_copy(src_s.at[pl.ds(base, 128)], sidx)
        pltpu.sync_copy(dst_s.at[pl.ds(base, 128)], didx)
        gets = [pltpu.async_copy(stbl.at[pl.ds(f * NP_S, NP_S)].at[sidx],
                                 rows.at[f], gsem[f]) for f in range(4)]
        puts = []
        for f in range(4):
            gets[f].wait()
            puts.append(pltpu.async_copy(
                rows.at[f], acc.at[pl.ds(f * NP_S, NP_S)].at[didx],
                ssem[f], add=True))
        for p in puts:
            p.wait()
        return carry

    lax.fori_loop(0, RB_S, body, 0)
    for r in range(RB_C):
        base = (s * RB_C + r) * 128
        pltpu.sync_copy(src_c.at[pl.ds(base, 128)], sidx)
        pltpu.sync_copy(dst_c.at[pl.ds(base, 128)], didx)
        gets = [pltpu.async_copy(stblc.at[pl.ds(f * NP_C, NP_C)].at[sidx],
                                 rows.at[f], gsem[f]) for f in range(4)]
        puts = []
        for f in range(4):
            gets[f].wait()
            puts.append(pltpu.async_copy(
                rows.at[f], accc.at[pl.ds(f * NP_C, NP_C)].at[didx],
                ssem[f], add=True))
        for p in puts:
            p.wait()
    plsc.subcore_barrier()
    for f in range(4):
        pltpu.sync_copy(acc.at[pl.ds(f * NP_S + s * TS_S, TS_S)],
                        agg.at[pl.ds((p0 + f) * NP_S + s * TS_S, TS_S)])
        pltpu.sync_copy(accc.at[pl.ds(f * NP_C + s * TS_C, TS_C)],
                        aggc.at[pl.ds((p0 + f) * NP_C + s * TS_C, TS_C)])


@functools.partial(
    pl.kernel,
    out_type=[_f32(64 * NP_S), _f32(64 * NP_C)],
    mesh=_MESH,
    scratch_types=[
        pltpu.VMEM_SHARED((16 * NP_S,), jnp.float32),
        pltpu.VMEM_SHARED((16 * NP_S,), jnp.float32),
        pltpu.VMEM_SHARED((16 * NP_C,), jnp.float32),
        pltpu.VMEM_SHARED((16 * NP_C,), jnp.float32),
        pltpu.VMEM((128,), jnp.int32),
        pltpu.VMEM((128,), jnp.int32),
        pltpu.VMEM((8, 128), jnp.float32),
        pltpu.SemaphoreType.DMA,
        pltpu.SemaphoreType.DMA,
        pltpu.SemaphoreType.DMA,
        pltpu.SemaphoreType.DMA,
        pltpu.SemaphoreType.DMA,
        pltpu.SemaphoreType.DMA,
        pltpu.SemaphoreType.DMA,
        pltpu.SemaphoreType.DMA,
        pltpu.SemaphoreType.DMA,
        pltpu.SemaphoreType.DMA,
        pltpu.SemaphoreType.DMA,
        pltpu.SemaphoreType.DMA,
        pltpu.SemaphoreType.DMA,
        pltpu.SemaphoreType.DMA,
        pltpu.SemaphoreType.DMA,
        pltpu.SemaphoreType.DMA,
    ],
)
def _sc_seg2(y2, y2c, src_s, dst_s, src_c, dst_c, zrow, agg, aggc,
             stbl, acc, stblc, accc, sidx, didx, rows,
             g0, g1, g2, g3, g4, g5, g6, g7,
             s0, s1, s2, s3, s4, s5, s6, s7):
    gsem = [g0, g1, g2, g3, g4, g5, g6, g7]
    ssem = [s0, s1, s2, s3, s4, s5, s6, s7]
    c = lax.axis_index("c")
    s = lax.axis_index("s")
    for g in range(2):
        p0 = c * 32 + g * 16
        for f in range(16):
            pltpu.sync_copy(y2.at[pl.ds((p0 + f) * NP_S + s * TS_S, TS_S)],
                            stbl.at[pl.ds(f * NP_S + s * TS_S, TS_S)])
            pltpu.sync_copy(zrow, acc.at[pl.ds(f * NP_S + s * TS_S, TS_S)])
            pltpu.sync_copy(y2c.at[pl.ds((p0 + f) * NP_C + s * TS_C, TS_C)],
                            stblc.at[pl.ds(f * NP_C + s * TS_C, TS_C)])
            pltpu.sync_copy(zrow.at[pl.ds(0, TS_C)],
                            accc.at[pl.ds(f * NP_C + s * TS_C, TS_C)])
        plsc.subcore_barrier()

        def body(r, carry):
            base = (s * RB_S + r) * 128
            pltpu.sync_copy(src_s.at[pl.ds(base, 128)], sidx)
            pltpu.sync_copy(dst_s.at[pl.ds(base, 128)], didx)

            def half(h, carry2):
                gets = [pltpu.async_copy(
                    stbl.at[pl.ds((h * 8 + f) * NP_S, NP_S)].at[sidx],
                    rows.at[f], gsem[f]) for f in range(8)]
                puts = []
                for f in range(8):
                    gets[f].wait()
                    puts.append(pltpu.async_copy(
                        rows.at[f],
                        acc.at[pl.ds((h * 8 + f) * NP_S, NP_S)].at[didx],
                        ssem[f], add=True))
                for p in puts:
                    p.wait()
                return carry2

            lax.fori_loop(0, 2, half, 0)
            return carry

        lax.fori_loop(0, RB_S, body, 0)
        for r in range(RB_C):
            base = (s * RB_C + r) * 128
            pltpu.sync_copy(src_c.at[pl.ds(base, 128)], sidx)
            pltpu.sync_copy(dst_c.at[pl.ds(base, 128)], didx)

            def chalf(h, carry2):
                gets = [pltpu.async_copy(
                    stblc.at[pl.ds((h * 8 + f) * NP_C, NP_C)].at[sidx],
                    rows.at[f], gsem[f]) for f in range(8)]
                puts = []
                for f in range(8):
                    gets[f].wait()
                    puts.append(pltpu.async_copy(
                        rows.at[f],
                        accc.at[pl.ds((h * 8 + f) * NP_C, NP_C)].at[didx],
                        ssem[f], add=True))
                for p in puts:
                    p.wait()
                return carry2

            lax.fori_loop(0, 2, chalf, 0)
        plsc.subcore_barrier()
        for f in range(16):
            pltpu.sync_copy(acc.at[pl.ds(f * NP_S + s * TS_S, TS_S)],
                            agg.at[pl.ds((p0 + f) * NP_S + s * TS_S, TS_S)])
            pltpu.sync_copy(accc.at[pl.ds(f * NP_C + s * TS_C, TS_C)],
                            aggc.at[pl.ds((p0 + f) * NP_C + s * TS_C, TS_C)])


def _tc_prep_body(deg_ref, xT_ref, dinv_ref, y1_ref):
    deg = deg_ref[0:1] + deg_ref[1:2] + 1.0
    dinv = lax.rsqrt(deg)
    dinv_ref[...] = dinv
    y1_ref[...] = dinv * xT_ref[...]


def _tc_prep(degp, xT, n_pad, blk):
    return pl.pallas_call(
        _tc_prep_body,
        grid=(n_pad // blk,),
        in_specs=[
            pl.BlockSpec((2, blk), lambda i: (0, i)),
            pl.BlockSpec((8, blk), lambda i: (0, i)),
        ],
        out_specs=[
            pl.BlockSpec((1, blk), lambda i: (0, i)),
            pl.BlockSpec((8, blk), lambda i: (0, i)),
        ],
        out_shape=[_f32(1, n_pad), _f32(8, n_pad)],
    )(degp, xT)


def _tc_l1_body(dinv_ref, agg_ref, y1_ref, w_ref, b_ref, y2_ref):
    pre = dinv_ref[...] * (agg_ref[...] + y1_ref[...])
    h = lax.dot_general(w_ref[...], pre, (((0,), (0,)), ((), ())),
                        preferred_element_type=jnp.float32)
    xs1 = jnp.maximum(h + b_ref[...], 0.0)
    y2_ref[...] = dinv_ref[...] * xs1


def _tc_l1(dinv, agg, y1, w_pad, b, n_pad, blk):
    return pl.pallas_call(
        _tc_l1_body,
        grid=(n_pad // blk,),
        in_specs=[
            pl.BlockSpec((1, blk), lambda i: (0, i)),
            pl.BlockSpec((8, blk), lambda i: (0, i)),
            pl.BlockSpec((8, blk), lambda i: (0, i)),
            pl.BlockSpec((8, 64), lambda i: (0, 0)),
            pl.BlockSpec((64, 1), lambda i: (0, 0)),
        ],
        out_specs=pl.BlockSpec((64, blk), lambda i: (0, i)),
        out_shape=_f32(64, n_pad),
    )(dinv, agg, y1, w_pad, b)


def _tc_l2c_body(dinv_ref, agg_ref, y2_ref, w_ref, b_ref, xc2_ref):
    pre = dinv_ref[...] * (agg_ref[...] + y2_ref[...])
    xc2 = jnp.maximum(
        lax.dot_general(pre, w_ref[...], (((0,), (0,)), ((), ())),
                        preferred_element_type=jnp.float32) + b_ref[...], 0.0)
    xc2_ref[...] = xc2[:512]


def _tc_l2c(dinv_c, agg2c, y2c, w2c, b2c):
    return pl.pallas_call(
        _tc_l2c_body,
        out_shape=_f32(512, 128),
    )(dinv_c, agg2c, y2c, w2c, b2c)


def _tc_final_body(dinv_ref, agg_ref, y2_ref, w2_ref, b2_ref, q_ref, xc2_ref,
                   aw1_ref, ab1_ref, aw2_ref, ab2_ref, cw_ref, cb_ref,
                   dw1_ref, db1_ref, dw2_ref, db2_ref,
                   cls_ref, dom_ref, fused_ref):
    pre = dinv_ref[...] * (agg_ref[...] + y2_ref[...])
    xs2 = jnp.maximum(
        lax.dot_general(pre, w2_ref[...], (((0,), (0,)), ((), ())),
                        preferred_element_type=jnp.float32) + b2_ref[...], 0.0)
    q = q_ref[...]
    qp = jnp.concatenate(
        [q, jnp.zeros((q.shape[0], 512 - N_C), jnp.float32)], axis=1)
    xc_dec = jnp.dot(qp, xc2_ref[...], preferred_element_type=jnp.float32)
    aw1 = aw1_ref[...]
    a = jnp.maximum(
        jnp.dot(xs2, aw1[:128], preferred_element_type=jnp.float32)
        + jnp.dot(xc_dec, aw1[128:], preferred_element_type=jnp.float32)
        + ab1_ref[...], 0.0)
    logits = jnp.dot(a, aw2_ref[...], preferred_element_type=jnp.float32) + ab2_ref[...]
    m = jnp.max(logits, axis=1, keepdims=True)
    e = jnp.exp(logits - m)
    wgt = e / jnp.sum(e, axis=1, keepdims=True)
    fused = wgt[:, 0:1] * xs2 + wgt[:, 1:2] * xc_dec
    cls_ref[...] = jnp.dot(fused, cw_ref[...],
                           preferred_element_type=jnp.float32) + cb_ref[...]
    d = jnp.maximum(jnp.dot(fused, dw1_ref[...],
                            preferred_element_type=jnp.float32) + db1_ref[...], 0.0)
    dom_ref[...] = jnp.dot(d, dw2_ref[...],
                           preferred_element_type=jnp.float32) + db2_ref[...]
    fused_ref[...] = fused


def _tc_final(dinv, agg2, y2, w2s, b2s, Q, xc2,
              attn_w1, attn_b1, attn_w2, attn_b2, cls_w, cls_b,
              dd_w1, dd_b1, dd_w2, dd_b2, blk):
    def full(shape):
        return pl.BlockSpec(shape, lambda i: tuple(0 for _ in shape))

    return pl.pallas_call(
        _tc_final_body,
        grid=((N_S + blk - 1) // blk,),
        in_specs=[
            pl.BlockSpec((1, blk), lambda i: (0, i)),
            pl.BlockSpec((64, blk), lambda i: (0, i)),
            pl.BlockSpec((64, blk), lambda i: (0, i)),
            full((64, 128)), full((1, 128)),
            pl.BlockSpec((blk, N_C), lambda i: (i, 0)),
            full((512, 128)),
            full((256, 128)), full((1, 128)),
            full((128, 2)), full((1, 2)),
            full((128, 9)), full((1, 9)),
            full((128, 64)), full((1, 64)),
            full((64, 2)), full((1, 2)),
        ],
        out_specs=[
            pl.BlockSpec((blk, 9), lambda i: (i, 0)),
            pl.BlockSpec((blk, 2), lambda i: (i, 0)),
            pl.BlockSpec((blk, 128), lambda i: (i, 0)),
        ],
        out_shape=[_f32(N_S, 9), _f32(N_S, 2), _f32(N_S, 128)],
    )(dinv, agg2, y2, w2s, b2s, Q, xc2,
      attn_w1, attn_b1, attn_w2, attn_b2, cls_w, cls_b,
      dd_w1, dd_b1, dd_w2, dd_b2)


def _pad_edges(edge_index, e_pad, n_real, n_pad):
    e_real = edge_index.shape[1]
    npad = e_pad - e_real
    njunk = n_pad - n_real
    fill_s = n_real + (jnp.arange(npad, dtype=jnp.int32) % njunk)
    fill_d = n_real + ((jnp.arange(npad, dtype=jnp.int32) * 7) % njunk)
    src = jnp.concatenate([edge_index[0], fill_s])
    dst = jnp.concatenate([edge_index[1], fill_d])
    return src, dst


def kernel(x_s, edge_index_s, x_c, edge_index_c, Q,
           w1s, b1s, w2s, b2s, w1c, b1c, w2c, b2c,
           attn_w1, attn_b1, attn_w2, attn_b2,
           cls_w, cls_b, dd_w1, dd_b1, dd_w2, dd_b2):
    f32 = jnp.float32
    src_s, dst_s = _pad_edges(edge_index_s, EP_S, N_S, NP_S)
    src_c, dst_c = _pad_edges(edge_index_c, EP_C, N_C, NP_C)
    xT_s = jnp.pad(x_s, ((0, NP_S - N_S), (0, 1))).T
    xT_c = jnp.pad(x_c, ((0, NP_C - N_C), (0, 1))).T
    w1s_pad = jnp.pad(w1s, ((0, 1), (0, 0)))
    w1c_pad = jnp.pad(w1c, ((0, 1), (0, 0)))
    zrow = jnp.zeros((TS_S,), f32)

    degp_s, degp_c = _sc_deg(dst_s, dst_c, zrow)

    dinv_s, y1s = _tc_prep(degp_s.reshape(2, NP_S), xT_s, NP_S, 6400)
    dinv_c, y1c = _tc_prep(degp_c.reshape(2, NP_C), xT_c, NP_C, NP_C)

    agg1, agg1c = _sc_seg1(y1s.reshape(-1), y1c.reshape(-1),
                           src_s, dst_s, src_c, dst_c, zrow)

    y2 = _tc_l1(dinv_s, agg1.reshape(8, NP_S), y1s, w1s_pad,
                b1s.reshape(64, 1), NP_S, 3200)
    y2c = _tc_l1(dinv_c, agg1c.reshape(8, NP_C), y1c, w1c_pad,
                 b1c.reshape(64, 1), NP_C, NP_C)

    agg2, agg2c = _sc_seg2(y2.reshape(-1), y2c.reshape(-1),
                           src_s, dst_s, src_c, dst_c, zrow)

    xc2 = _tc_l2c(dinv_c, agg2c.reshape(64, NP_C), y2c, w2c,
                  b2c.reshape(1, -1))

    return _tc_final(dinv_s, agg2.reshape(64, NP_S), y2, w2s,
                     b2s.reshape(1, -1), Q, xc2,
                     attn_w1, attn_b1.reshape(1, -1),
                     attn_w2, attn_b2.reshape(1, -1),
                     cls_w, cls_b.reshape(1, -1),
                     dd_w1, dd_b1.reshape(1, -1),
                     dd_w2, dd_b2.reshape(1, -1), 1024)


# speedup vs baseline: 10.1286x; 1.2724x over previous
import functools

import jax
import jax.numpy as jnp
from jax import lax
from jax.experimental import pallas as pl
from jax.experimental.pallas import tpu as pltpu
from jax.experimental.pallas import tpu_sc as plsc

N_S, N_C = 50000, 500
E_S, E_C = 800000, 8000
NP_S, NP_C = 51200, 2048
EP_S, EP_C = 802816, 8192
RS, RC = EP_S // 128, EP_C // 128
TS_S, TS_C = NP_S // 16, NP_C // 16
RW_S, RW_C = RS // 32, RC // 32
RB_S, RB_C = RS // 16, RC // 16

_MESH = plsc.VectorSubcoreMesh(core_axis_name="c", subcore_axis_name="s")


def _f32(*shape):
    return jax.ShapeDtypeStruct(shape, jnp.float32)


@functools.partial(
    pl.kernel,
    out_type=[_f32(2 * NP_S), _f32(2 * NP_C)],
    mesh=_MESH,
    scratch_types=[
        pltpu.VMEM_SHARED((NP_S,), jnp.float32),
        pltpu.VMEM_SHARED((NP_C,), jnp.float32),
        pltpu.VMEM((128,), jnp.int32),
        pltpu.VMEM((128,), jnp.float32),
    ],
)
def _sc_deg(dst_s, dst_c, zrow, deg_s, deg_c, acc_s, acc_c, didx, ones_v):
    c = lax.axis_index("c")
    s = lax.axis_index("s")
    w = c * 16 + s
    for j in range(8):
        ones_v[pl.ds(j * 16, 16)] = jnp.ones((16,), jnp.float32)
    pltpu.sync_copy(zrow, acc_s.at[pl.ds(s * TS_S, TS_S)])
    pltpu.sync_copy(zrow.at[pl.ds(0, TS_C)], acc_c.at[pl.ds(s * TS_C, TS_C)])
    plsc.subcore_barrier()

    def body(r, carry):
        pltpu.sync_copy(dst_s.at[pl.ds((w * RW_S + r) * 128, 128)], didx)
        pltpu.sync_copy(ones_v, acc_s.at[didx], add=True)
        return carry

    lax.fori_loop(0, RW_S, body, 0)
    for r in range(RW_C):
        pltpu.sync_copy(dst_c.at[pl.ds((w * RW_C + r) * 128, 128)], didx)
        pltpu.sync_copy(ones_v, acc_c.at[didx], add=True)
    plsc.subcore_barrier()
    pltpu.sync_copy(acc_s.at[pl.ds(s * TS_S, TS_S)],
                    deg_s.at[pl.ds(c * NP_S + s * TS_S, TS_S)])
    pltpu.sync_copy(acc_c.at[pl.ds(s * TS_C, TS_C)],
                    deg_c.at[pl.ds(c * NP_C + s * TS_C, TS_C)])


@functools.partial(
    pl.kernel,
    out_type=[_f32(8 * NP_S), _f32(8 * NP_C)],
    mesh=_MESH,
    scratch_types=[
        pltpu.VMEM_SHARED((4 * NP_S,), jnp.float32),
        pltpu.VMEM_SHARED((4 * NP_S,), jnp.float32),
        pltpu.VMEM_SHARED((4 * NP_C,), jnp.float32),
        pltpu.VMEM_SHARED((4 * NP_C,), jnp.float32),
        pltpu.VMEM((16, 128), jnp.int32),
        pltpu.VMEM((4, 128), jnp.float32),
        pltpu.SemaphoreType.DMA,
        pltpu.SemaphoreType.DMA,
        pltpu.SemaphoreType.DMA,
        pltpu.SemaphoreType.DMA,
        pltpu.SemaphoreType.DMA,
        pltpu.SemaphoreType.DMA,
        pltpu.SemaphoreType.DMA,
        pltpu.SemaphoreType.DMA,
    ],
)
def _sc_seg1(y1, y1c, es, ec, zrow, agg, aggc,
             stbl, acc, stblc, accc, idxb, rows,
             gs0, gs1, gs2, gs3, ss0, ss1, ss2, ss3):
    gsem = [gs0, gs1, gs2, gs3]
    ssem = [ss0, ss1, ss2, ss3]
    c = lax.axis_index("c")
    s = lax.axis_index("s")
    p0 = c * 4
    for f in range(4):
        pltpu.sync_copy(y1.at[pl.ds((p0 + f) * NP_S + s * TS_S, TS_S)],
                        stbl.at[pl.ds(f * NP_S + s * TS_S, TS_S)])
        pltpu.sync_copy(zrow, acc.at[pl.ds(f * NP_S + s * TS_S, TS_S)])
        pltpu.sync_copy(y1c.at[pl.ds((p0 + f) * NP_C + s * TS_C, TS_C)],
                        stblc.at[pl.ds(f * NP_C + s * TS_C, TS_C)])
        pltpu.sync_copy(zrow.at[pl.ds(0, TS_C)],
                        accc.at[pl.ds(f * NP_C + s * TS_C, TS_C)])
    plsc.subcore_barrier()

    def outer(r8, carry):
        pltpu.sync_copy(es.at[pl.ds((s * RB_S + r8 * 8) * 2, 16), :], idxb)

        def body(k, carry2):
            gets = [pltpu.async_copy(
                stbl.at[pl.ds(f * NP_S, NP_S)].at[idxb.at[2 * k]],
                rows.at[f], gsem[f]) for f in range(4)]
            puts = []
            for f in range(4):
                gets[f].wait()
                puts.append(pltpu.async_copy(
                    rows.at[f],
                    acc.at[pl.ds(f * NP_S, NP_S)].at[idxb.at[2 * k + 1]],
                    ssem[f], add=True))
            for p in puts:
                p.wait()
            return carry2

        lax.fori_loop(0, 8, body, 0)
        return carry

    lax.fori_loop(0, RB_S // 8, outer, 0)
    pltpu.sync_copy(ec.at[pl.ds(s * RB_C * 2, 8), :], idxb.at[pl.ds(0, 8)])
    for r in range(RB_C):
        gets = [pltpu.async_copy(
            stblc.at[pl.ds(f * NP_C, NP_C)].at[idxb.at[2 * r]],
            rows.at[f], gsem[f]) for f in range(4)]
        puts = []
        for f in range(4):
            gets[f].wait()
            puts.append(pltpu.async_copy(
                rows.at[f],
                accc.at[pl.ds(f * NP_C, NP_C)].at[idxb.at[2 * r + 1]],
                ssem[f], add=True))
        for p in puts:
            p.wait()
    plsc.subcore_barrier()
    for f in range(4):
        pltpu.sync_copy(acc.at[pl.ds(f * NP_S + s * TS_S, TS_S)],
                        agg.at[pl.ds((p0 + f) * NP_S + s * TS_S, TS_S)])
        pltpu.sync_copy(accc.at[pl.ds(f * NP_C + s * TS_C, TS_C)],
                        aggc.at[pl.ds((p0 + f) * NP_C + s * TS_C, TS_C)])


@functools.partial(
    pl.kernel,
    out_type=[_f32(64 * NP_S), _f32(64 * NP_C)],
    mesh=_MESH,
    scratch_types=[
        pltpu.VMEM_SHARED((16 * NP_S,), jnp.float32),
        pltpu.VMEM_SHARED((16 * NP_S,), jnp.float32),
        pltpu.VMEM_SHARED((16 * NP_C,), jnp.float32),
        pltpu.VMEM_SHARED((16 * NP_C,), jnp.float32),
        pltpu.VMEM((16, 128), jnp.int32),
        pltpu.VMEM((8, 128), jnp.float32),
        pltpu.SemaphoreType.DMA,
        pltpu.SemaphoreType.DMA,
        pltpu.SemaphoreType.DMA,
        pltpu.SemaphoreType.DMA,
        pltpu.SemaphoreType.DMA,
        pltpu.SemaphoreType.DMA,
        pltpu.SemaphoreType.DMA,
        pltpu.SemaphoreType.DMA,
        pltpu.SemaphoreType.DMA,
        pltpu.SemaphoreType.DMA,
        pltpu.SemaphoreType.DMA,
        pltpu.SemaphoreType.DMA,
        pltpu.SemaphoreType.DMA,
        pltpu.SemaphoreType.DMA,
        pltpu.SemaphoreType.DMA,
        pltpu.SemaphoreType.DMA,
    ],
)
def _sc_seg2(y2, y2c, es, ec, zrow, agg, aggc,
             stbl, acc, stblc, accc, idxb, rows,
             g0, g1, g2, g3, g4, g5, g6, g7,
             s0, s1, s2, s3, s4, s5, s6, s7):
    gsem = [g0, g1, g2, g3, g4, g5, g6, g7]
    ssem = [s0, s1, s2, s3, s4, s5, s6, s7]
    c = lax.axis_index("c")
    s = lax.axis_index("s")
    for g in range(2):
        p0 = c * 32 + g * 16
        for f in range(16):
            pltpu.sync_copy(y2.at[pl.ds((p0 + f) * NP_S + s * TS_S, TS_S)],
                            stbl.at[pl.ds(f * NP_S + s * TS_S, TS_S)])
            pltpu.sync_copy(zrow, acc.at[pl.ds(f * NP_S + s * TS_S, TS_S)])
            pltpu.sync_copy(y2c.at[pl.ds((p0 + f) * NP_C + s * TS_C, TS_C)],
                            stblc.at[pl.ds(f * NP_C + s * TS_C, TS_C)])
            pltpu.sync_copy(zrow.at[pl.ds(0, TS_C)],
                            accc.at[pl.ds(f * NP_C + s * TS_C, TS_C)])
        plsc.subcore_barrier()

        def outer(r8, carry):
            pltpu.sync_copy(es.at[pl.ds((s * RB_S + r8 * 8) * 2, 16), :], idxb)

            def body(k, carry2):
                def half(h, carry3):
                    gets = [pltpu.async_copy(
                        stbl.at[pl.ds((h * 8 + f) * NP_S, NP_S)]
                        .at[idxb.at[2 * k]],
                        rows.at[f], gsem[f]) for f in range(8)]
                    puts = []
                    for f in range(8):
                        gets[f].wait()
                        puts.append(pltpu.async_copy(
                            rows.at[f],
                            acc.at[pl.ds((h * 8 + f) * NP_S, NP_S)]
                            .at[idxb.at[2 * k + 1]],
                            ssem[f], add=True))
                    for p in puts:
                        p.wait()
                    return carry3

                lax.fori_loop(0, 2, half, 0)
                return carry2

            lax.fori_loop(0, 8, body, 0)
            return carry

        lax.fori_loop(0, RB_S // 8, outer, 0)
        pltpu.sync_copy(ec.at[pl.ds(s * RB_C * 2, 8), :], idxb.at[pl.ds(0, 8)])
        for r in range(RB_C):

            def chalf(h, carry2):
                gets = [pltpu.async_copy(
                    stblc.at[pl.ds((h * 8 + f) * NP_C, NP_C)]
                    .at[idxb.at[2 * r]],
                    rows.at[f], gsem[f]) for f in range(8)]
                puts = []
                for f in range(8):
                    gets[f].wait()
                    puts.append(pltpu.async_copy(
                        rows.at[f],
                        accc.at[pl.ds((h * 8 + f) * NP_C, NP_C)]
                        .at[idxb.at[2 * r + 1]],
                        ssem[f], add=True))
                for p in puts:
                    p.wait()
                return carry2

            lax.fori_loop(0, 2, chalf, 0)
        plsc.subcore_barrier()
        for f in range(16):
            pltpu.sync_copy(acc.at[pl.ds(f * NP_S + s * TS_S, TS_S)],
                            agg.at[pl.ds((p0 + f) * NP_S + s * TS_S, TS_S)])
            pltpu.sync_copy(accc.at[pl.ds(f * NP_C + s * TS_C, TS_C)],
                            aggc.at[pl.ds((p0 + f) * NP_C + s * TS_C, TS_C)])


def _tc_prep_body(deg_ref, xT_ref, dinv_ref, y1_ref):
    deg = deg_ref[0:1] + deg_ref[1:2] + 1.0
    dinv = lax.rsqrt(deg)
    dinv_ref[...] = dinv
    y1_ref[...] = dinv * xT_ref[...]


def _tc_prep(degp, xT, n_pad, blk):
    return pl.pallas_call(
        _tc_prep_body,
        grid=(n_pad // blk,),
        in_specs=[
            pl.BlockSpec((2, blk), lambda i: (0, i)),
            pl.BlockSpec((8, blk), lambda i: (0, i)),
        ],
        out_specs=[
            pl.BlockSpec((1, blk), lambda i: (0, i)),
            pl.BlockSpec((8, blk), lambda i: (0, i)),
        ],
        out_shape=[_f32(1, n_pad), _f32(8, n_pad)],
    )(degp, xT)


def _tc_l1_body(dinv_ref, agg_ref, y1_ref, w_ref, b_ref, y2_ref):
    pre = dinv_ref[...] * (agg_ref[...] + y1_ref[...])
    h = lax.dot_general(w_ref[...], pre, (((0,), (0,)), ((), ())),
                        preferred_element_type=jnp.float32)
    xs1 = jnp.maximum(h + b_ref[...], 0.0)
    y2_ref[...] = dinv_ref[...] * xs1


def _tc_l1(dinv, agg, y1, w_pad, b, n_pad, blk):
    return pl.pallas_call(
        _tc_l1_body,
        grid=(n_pad // blk,),
        in_specs=[
            pl.BlockSpec((1, blk), lambda i: (0, i)),
            pl.BlockSpec((8, blk), lambda i: (0, i)),
            pl.BlockSpec((8, blk), lambda i: (0, i)),
            pl.BlockSpec((8, 64), lambda i: (0, 0)),
            pl.BlockSpec((64, 1), lambda i: (0, 0)),
        ],
        out_specs=pl.BlockSpec((64, blk), lambda i: (0, i)),
        out_shape=_f32(64, n_pad),
    )(dinv, agg, y1, w_pad, b)


def _tc_l2c_body(dinv_ref, agg_ref, y2_ref, w_ref, b_ref, xc2_ref):
    pre = dinv_ref[...] * (agg_ref[...] + y2_ref[...])
    xc2 = jnp.maximum(
        lax.dot_general(pre, w_ref[...], (((0,), (0,)), ((), ())),
                        preferred_element_type=jnp.float32) + b_ref[...], 0.0)
    xc2_ref[...] = xc2[:512]


def _tc_l2c(dinv_c, agg2c, y2c, w2c, b2c):
    return pl.pallas_call(
        _tc_l2c_body,
        out_shape=_f32(512, 128),
    )(dinv_c, agg2c, y2c, w2c, b2c)


def _tc_final_body(dinv_ref, agg_ref, y2_ref, w2_ref, b2_ref, q_ref, xc2_ref,
                   aw1_ref, ab1_ref, aw2_ref, ab2_ref, cw_ref, cb_ref,
                   dw1_ref, db1_ref, dw2_ref, db2_ref,
                   cls_ref, dom_ref, fused_ref):
    pre = dinv_ref[...] * (agg_ref[...] + y2_ref[...])
    xs2 = jnp.maximum(
        lax.dot_general(pre, w2_ref[...], (((0,), (0,)), ((), ())),
                        preferred_element_type=jnp.float32) + b2_ref[...], 0.0)
    q = q_ref[...]
    qp = jnp.concatenate(
        [q, jnp.zeros((q.shape[0], 512 - N_C), jnp.float32)], axis=1)
    xc_dec = jnp.dot(qp, xc2_ref[...], preferred_element_type=jnp.float32)
    aw1 = aw1_ref[...]
    a = jnp.maximum(
        jnp.dot(xs2, aw1[:128], preferred_element_type=jnp.float32)
        + jnp.dot(xc_dec, aw1[128:], preferred_element_type=jnp.float32)
        + ab1_ref[...], 0.0)
    logits = jnp.dot(a, aw2_ref[...], preferred_element_type=jnp.float32) + ab2_ref[...]
    m = jnp.max(logits, axis=1, keepdims=True)
    e = jnp.exp(logits - m)
    wgt = e / jnp.sum(e, axis=1, keepdims=True)
    fused = wgt[:, 0:1] * xs2 + wgt[:, 1:2] * xc_dec
    cls_ref[...] = jnp.dot(fused, cw_ref[...],
                           preferred_element_type=jnp.float32) + cb_ref[...]
    d = jnp.maximum(jnp.dot(fused, dw1_ref[...],
                            preferred_element_type=jnp.float32) + db1_ref[...], 0.0)
    dom_ref[...] = jnp.dot(d, dw2_ref[...],
                           preferred_element_type=jnp.float32) + db2_ref[...]
    fused_ref[...] = fused


def _tc_final(dinv, agg2, y2, w2s, b2s, Q, xc2,
              attn_w1, attn_b1, attn_w2, attn_b2, cls_w, cls_b,
              dd_w1, dd_b1, dd_w2, dd_b2, blk):
    def full(shape):
        return pl.BlockSpec(shape, lambda i: tuple(0 for _ in shape))

    return pl.pallas_call(
        _tc_final_body,
        grid=((N_S + blk - 1) // blk,),
        in_specs=[
            pl.BlockSpec((1, blk), lambda i: (0, i)),
            pl.BlockSpec((64, blk), lambda i: (0, i)),
            pl.BlockSpec((64, blk), lambda i: (0, i)),
            full((64, 128)), full((1, 128)),
            pl.BlockSpec((blk, N_C), lambda i: (i, 0)),
            full((512, 128)),
            full((256, 128)), full((1, 128)),
            full((128, 2)), full((1, 2)),
            full((128, 9)), full((1, 9)),
            full((128, 64)), full((1, 64)),
            full((64, 2)), full((1, 2)),
        ],
        out_specs=[
            pl.BlockSpec((blk, 9), lambda i: (i, 0)),
            pl.BlockSpec((blk, 2), lambda i: (i, 0)),
            pl.BlockSpec((blk, 128), lambda i: (i, 0)),
        ],
        out_shape=[_f32(N_S, 9), _f32(N_S, 2), _f32(N_S, 128)],
    )(dinv, agg2, y2, w2s, b2s, Q, xc2,
      attn_w1, attn_b1, attn_w2, attn_b2, cls_w, cls_b,
      dd_w1, dd_b1, dd_w2, dd_b2)


def _pad_edges(edge_index, e_pad, n_real, n_pad):
    e_real = edge_index.shape[1]
    npad = e_pad - e_real
    njunk = n_pad - n_real
    fill_s = n_real + (jnp.arange(npad, dtype=jnp.int32) % njunk)
    fill_d = n_real + ((jnp.arange(npad, dtype=jnp.int32) * 7) % njunk)
    src = jnp.concatenate([edge_index[0], fill_s])
    dst = jnp.concatenate([edge_index[1], fill_d])
    return src, dst


def kernel(x_s, edge_index_s, x_c, edge_index_c, Q,
           w1s, b1s, w2s, b2s, w1c, b1c, w2c, b2c,
           attn_w1, attn_b1, attn_w2, attn_b2,
           cls_w, cls_b, dd_w1, dd_b1, dd_w2, dd_b2):
    f32 = jnp.float32
    src_s, dst_s = _pad_edges(edge_index_s, EP_S, N_S, NP_S)
    src_c, dst_c = _pad_edges(edge_index_c, EP_C, N_C, NP_C)
    xT_s = jnp.pad(x_s, ((0, NP_S - N_S), (0, 1))).T
    xT_c = jnp.pad(x_c, ((0, NP_C - N_C), (0, 1))).T
    w1s_pad = jnp.pad(w1s, ((0, 1), (0, 0)))
    w1c_pad = jnp.pad(w1c, ((0, 1), (0, 0)))
    zrow = jnp.zeros((TS_S,), f32)

    degp_s, degp_c = _sc_deg(dst_s, dst_c, zrow)

    dinv_s, y1s = _tc_prep(degp_s.reshape(2, NP_S), xT_s, NP_S, 6400)
    dinv_c, y1c = _tc_prep(degp_c.reshape(2, NP_C), xT_c, NP_C, NP_C)

    es_s = jnp.stack([src_s.reshape(RS, 128), dst_s.reshape(RS, 128)],
                     axis=1).reshape(RS * 2, 128)
    es_c = jnp.stack([src_c.reshape(RC, 128), dst_c.reshape(RC, 128)],
                     axis=1).reshape(RC * 2, 128)

    agg1, agg1c = _sc_seg1(y1s.reshape(-1), y1c.reshape(-1), es_s, es_c, zrow)

    y2 = _tc_l1(dinv_s, agg1.reshape(8, NP_S), y1s, w1s_pad,
                b1s.reshape(64, 1), NP_S, 3200)
    y2c = _tc_l1(dinv_c, agg1c.reshape(8, NP_C), y1c, w1c_pad,
                 b1c.reshape(64, 1), NP_C, NP_C)

    agg2, agg2c = _sc_seg2(y2.reshape(-1), y2c.reshape(-1), es_s, es_c, zrow)

    xc2 = _tc_l2c(dinv_c, agg2c.reshape(64, NP_C), y2c, w2c,
                  b2c.reshape(1, -1))

    return _tc_final(dinv_s, agg2.reshape(64, NP_S), y2, w2s,
                     b2s.reshape(1, -1), Q, xc2,
                     attn_w1, attn_b1.reshape(1, -1),
                     attn_w2, attn_b2.reshape(1, -1),
                     cls_w, cls_b.reshape(1, -1),
                     dd_w1, dd_b1.reshape(1, -1),
                     dd_w2, dd_b2.reshape(1, -1), 1024)
